# baseline (speedup 1.0000x reference)
import functools

import jax
import jax.numpy as jnp
from jax import lax
from jax.experimental import pallas as pl
from jax.experimental.pallas import tpu as pltpu
from jax.experimental.pallas import tpu_sc as plsc

VOCAB = 100000
EMBED = 64
BATCH = 16384
SEQ = 20
SEQ_PAD = 24
LANE_PAD = 128
LANES = 16

NUM_CORES = 2
NUM_SUBCORES = 16
NW = NUM_CORES * NUM_SUBCORES

TOTAL = BATCH * SEQ
B_PER_W = TOTAL // NW
BATCH_PER_W = BATCH // NW
CHUNK_B = 32
CHUNK = CHUNK_B * SEQ
NSTEP = BATCH_PER_W // CHUNK_B
NBUF = 2

assert BATCH % NW == 0 and BATCH_PER_W % CHUNK_B == 0 and CHUNK % 8 == 0

_mesh = plsc.VectorSubcoreMesh(core_axis_name="c", subcore_axis_name="s")


@functools.partial(
    pl.kernel,
    mesh=_mesh,
    out_type=jax.ShapeDtypeStruct((BATCH, SEQ_PAD, LANE_PAD), jnp.float32),
    scratch_types=[
        pltpu.VMEM((CHUNK_B, LANE_PAD), jnp.int32),
        [pltpu.VMEM((CHUNK + LANES, ), jnp.int32) for _ in range(NBUF)],
        [pltpu.VMEM((CHUNK, EMBED), jnp.float32) for _ in range(NBUF)],
        [pltpu.SemaphoreType.DMA for _ in range(NBUF)],
        [pltpu.SemaphoreType.DMA for _ in range(NBUF)],
    ],
    compiler_params=pltpu.CompilerParams(use_tc_tiling_on_sc=False),
)
def _gather_kernel(table_hbm, q_hbm, out_hbm, qpage, idxb, rows, gsem, osem):
    wid = lax.axis_index("s") * NUM_CORES + lax.axis_index("c")
    batch_base = wid * BATCH_PER_W

    def build_idx(g):
        buf = idxb[g % NBUF]
        pltpu.sync_copy(q_hbm.at[pl.ds(batch_base + g * CHUNK_B, CHUNK_B)],
                        qpage)
        for j in range(CHUNK_B):
            head = qpage[j, pl.ds(0, LANES)]
            tail = qpage[j, pl.ds(LANES, LANES)]
            buf[pl.ds(j * SEQ, LANES)] = head
            buf[pl.ds(j * SEQ + LANES, LANES)] = tail

    def fire_gather(g):
        idx_slice = idxb[g % NBUF].at[pl.ds(0, CHUNK)]
        return pltpu.async_copy(table_hbm.at[idx_slice], rows[g % NBUF],
                                gsem[g % NBUF])

    def fire_writes(g):
        buf = rows[g % NBUF]
        sem = osem[g % NBUF]
        copies = []
        for j in range(CHUNK_B):
            b = batch_base + g * CHUNK_B + j
            dst = out_hbm.at[b, pl.ds(0, SEQ), pl.ds(0, EMBED)]
            copies.append(
                pltpu.async_copy(buf.at[pl.ds(j * SEQ, SEQ)], dst, sem))
        return copies

    build_idx(0)
    gathers = {0: fire_gather(0)}
    writes = {}
    for g in range(NSTEP):
        if g + 1 < NSTEP:
            prev_w = g + 1 - NBUF
            if prev_w >= 0:
                for c in writes.pop(prev_w):
                    c.wait()
            build_idx(g + 1)
            gathers[g + 1] = fire_gather(g + 1)
        gathers.pop(g).wait()
        writes[g] = fire_writes(g)
    for g in sorted(writes):
        for c in writes.pop(g):
            c.wait()


def kernel(q, q_len, table):
    del q_len
    q2 = jnp.pad(q.astype(jnp.int32) * 2, ((0, 0), (0, LANE_PAD - SEQ)))
    table2 = jnp.pad(table, ((0, 0), (0, LANE_PAD - EMBED))).reshape(
        2 * VOCAB, EMBED)
    padded = _gather_kernel(table2, q2)
    return padded[:, :SEQ, :EMBED]
